# baseline (speedup 1.0000x reference)
import jax
import jax.numpy as jnp
from jax import lax
from jax.experimental import pallas as pl
from jax.experimental.pallas import tpu as pltpu
from jax.experimental.pallas import tpu_sc as plsc

N = 900
NN = N * N
NG = 10
NPG = 90
F1 = 90
E = 28800

NS = 16
EPT = 1800
EBUF = 1808
SLOTS = 1920
E2 = SLOTS * NS
CH = SLOTS // 128
PT_A = 50640
PT_STRIDE = 50624
ZCH = 4096
RCH = 16880


def _adj_body(edges, out, src_v, dst_v, flat_v, ones_v, zero_v, acc,
              semz, seme, semr):
    s = lax.axis_index("s")

    base = s * EPT
    eh0 = pltpu.async_copy(edges.at[pl.ds(base, EPT)],
                           src_v.at[pl.ds(0, EPT)], seme)
    eh1 = pltpu.async_copy(edges.at[pl.ds(E + base, EPT)],
                           dst_v.at[pl.ds(0, EPT)], seme)

    def fill_zero(i, _):
        zero_v[pl.ds(i * 16, 16)] = jnp.zeros((16,), jnp.float32)
        return 0

    lax.fori_loop(0, ZCH // 16, fill_zero, 0)
    for k in range(128 // 16):
        ones_v[pl.ds(k * 16, 16)] = jnp.full((16,), 1.0, jnp.float32)

    zh = []
    for q in range(12):
        zh.append(pltpu.async_copy(
            zero_v.at[pl.ds(0, ZCH)],
            acc.at[pl.ds(s * PT_STRIDE + q * ZCH, ZCH)], semz))
    zh.append(pltpu.async_copy(
        zero_v.at[pl.ds(0, PT_A - 12 * ZCH)],
        acc.at[pl.ds(s * PT_STRIDE + 12 * ZCH, PT_A - 12 * ZCH)], semz))

    eh0.wait()
    eh1.wait()

    nfull = EPT // 16
    for j in range(CH):
        def fill_flat(k, _, j=j):
            off = j * 128 + k * 16
            fl = dst_v[pl.ds(off, 16)] * N + src_v[pl.ds(off, 16)]
            flat_v[j, pl.ds(k * 16, 16)] = fl
            return 0

        def fill_pad(k, _, j=j):
            flat_v[j, pl.ds(k * 16, 16)] = jnp.zeros((16,), jnp.int32)
            return 0

        lo = j * 8
        if (j + 1) * 8 <= nfull:
            lax.fori_loop(0, 8, fill_flat, 0)
        else:
            for k in range(8):
                g = lo + k
                if g < nfull:
                    fill_flat(k, 0)
                elif g == nfull:
                    off = g * 16
                    fl = dst_v[pl.ds(off, 16)] * N + src_v[pl.ds(off, 16)]
                    lane = lax.iota(jnp.int32, 16)
                    fl = jnp.where(lane < EPT - nfull * 16, fl, 0)
                    flat_v[j, pl.ds(k * 16, 16)] = fl
                else:
                    fill_pad(k, 0)

    for h in zh:
        h.wait()
    plsc.subcore_barrier()
    sh = []
    for j in range(CH):
        sh.append(pltpu.async_copy(ones_v, acc.at[flat_v.at[j]], seme,
                                   add=True))
    for h in sh:
        h.wait()
    plsc.subcore_barrier()

    rh = []
    for c in range(3):
        pltpu.sync_copy(acc.at[pl.ds(s * PT_STRIDE + c * RCH, RCH)],
                        zero_v.at[pl.ds(c * RCH, RCH)])
        rh.append(pltpu.async_copy(
            zero_v.at[pl.ds(c * RCH, RCH)],
            out.at[pl.ds(s * PT_STRIDE + c * RCH, RCH)], semr))
    for h in rh:
        h.wait()


def _adj_call(ei):
    k = pl.kernel(
        _adj_body,
        out_type=jax.ShapeDtypeStruct((NN,), jnp.float32),
        mesh=plsc.VectorSubcoreMesh(core_axis_name="c", subcore_axis_name="s",
                                    num_cores=1),
        scratch_types=[
            pltpu.VMEM((EBUF,), jnp.int32),
            pltpu.VMEM((EBUF,), jnp.int32),
            pltpu.VMEM((CH, 128), jnp.int32),
            pltpu.VMEM((128,), jnp.float32),
            pltpu.VMEM((PT_A,), jnp.float32),
            pltpu.VMEM_SHARED((NN,), jnp.float32),
            pltpu.SemaphoreType.DMA,
            pltpu.SemaphoreType.DMA,
            pltpu.SemaphoreType.DMA,
        ],
    )
    return k(ei)


def _dense_body(p_ref, x_ref, batch_ref, w1_ref, b1_ref, w2_ref, b2_ref,
                out_ref):
    A = p_ref[:]
    padc = float(E2 - E)
    row0 = jnp.where(
        lax.broadcasted_iota(jnp.int32, (N, 1), 0) == 0, 1.0, 0.0)
    deg = jnp.sum(A, axis=1, keepdims=True) + 1.0 - padc * row0
    dinv = lax.rsqrt(deg)

    g1 = dinv * jnp.dot(x_ref[:], w1_ref[:], preferred_element_type=jnp.float32)
    a1 = jnp.dot(A, g1, preferred_element_type=jnp.float32) \
        - row0 * (padc * g1[0:1, :])
    h1 = dinv * (a1 + g1) + b1_ref[:]
    h1 = jnp.maximum(h1, 0.0)

    ssum = h1[0:NPG, :]
    for b in range(1, NG):
        ssum = ssum + h1[b * NPG:(b + 1) * NPG, :]
    sm = ssum * (1.0 / NG)
    sm = sm - jnp.max(sm, axis=1, keepdims=True)
    es = jnp.exp(sm)
    s_soft = es / jnp.sum(es, axis=1, keepdims=True)

    blocks = []
    for b in range(NG):
        hb = h1[b * NPG:(b + 1) * NPG, :]
        blocks.append(
            lax.dot_general(s_soft, hb, (((0,), (0,)), ((), ())),
                            preferred_element_type=jnp.float32))
    h2 = jnp.concatenate(blocks, axis=0)

    g2 = dinv * jnp.dot(h2, w2_ref[:], preferred_element_type=jnp.float32)
    a2 = jnp.dot(A, g2, preferred_element_type=jnp.float32) \
        - row0 * (padc * g2[0:1, :])
    h3 = dinv * (a2 + g2) + b2_ref[:]

    gi = lax.broadcasted_iota(jnp.int32, (NG, N), 0)
    bmat = jnp.where(batch_ref[:] == gi, 1.0, 0.0)
    cnt = jnp.sum(bmat, axis=1, keepdims=True)
    gm = jnp.dot(bmat, h3, preferred_element_type=jnp.float32)
    gm = gm / jnp.maximum(cnt, 1.0)

    z = gm - jnp.max(gm, axis=1, keepdims=True)
    out_ref[:] = z - jnp.log(jnp.sum(jnp.exp(z), axis=1, keepdims=True))


def _dense_call(P, x, batch2d, W1, b1, W2, b2):
    return pl.pallas_call(
        _dense_body,
        out_shape=jax.ShapeDtypeStruct((NG, 4), jnp.float32),
    )(P, x, batch2d, W1, b1, W2, b2)


def kernel(x, adj, edge_index, batch, W1, b1, W2, b2):
    del adj
    ei = edge_index.astype(jnp.int32).reshape(-1)
    P = _adj_call(ei).reshape(N, N)
    return _dense_call(P, x, batch.astype(jnp.int32).reshape(1, N), W1,
                       b1.reshape(1, F1), W2, b2.reshape(1, 4))
